# baseline (speedup 1.0000x reference)
import functools

import jax
import jax.numpy as jnp
import numpy as np
from jax import lax
from jax.experimental import pallas as pl
from jax.experimental.pallas import tpu as pltpu
from jax.experimental.pallas import tpu_sc as plsc

N_NODES = 10000
N_EDGES = 160000
D = 256

NC = 2
NS = 16
NW = NC * NS
ROWS = 320
TRASH = ROWS
SLAB = ROWS + 8
SENT = N_NODES
SCE = 2000
NSC = N_EDGES // SCE
GB = 80
LOGCAP = SCE + GB + 16
BIG = np.int32(1 << 20)

_GDN = lax.GatherDimensionNumbers(offset_dims=(), collapsed_slice_dims=(0,),
                                  start_index_map=(0,))


def _take16(v, idx):
    return lax.gather(v, idx.reshape(16, 1), _GDN, (1,),
                      mode=lax.GatherScatterMode.PROMISE_IN_BOUNDS)


def _bmin(v, perms):
    for p in perms:
        v = jnp.minimum(v, _take16(v, p))
    return v


def _bsum(v, perms):
    for p in perms:
        v = v + _take16(v, p)
    return v


def _sc_agg_build():
    mesh = plsc.VectorSubcoreMesh(core_axis_name="c", subcore_axis_name="s")

    @functools.partial(
        pl.kernel,
        mesh=mesh,
        out_type=jax.ShapeDtypeStruct((NW * ROWS, D), jnp.float32),
        scratch_types=[
            pltpu.VMEM((SCE,), jnp.int32),
            pltpu.VMEM((SCE,), jnp.int32),
            pltpu.VMEM((LOGCAP,), jnp.int32),
            pltpu.VMEM((LOGCAP,), jnp.int32),
            pltpu.VMEM((GB, D), jnp.float32),
            pltpu.VMEM((SLAB, D), jnp.float32),
            pltpu.VMEM((16,), jnp.int32),
            pltpu.VMEM((16,), jnp.int32),
            pltpu.SemaphoreType.DMA,
        ],
    )
    def sc_agg(x_hbm, src_hbm, dst_hbm, agg_hbm,
               srcbuf, dstbuf, logsrc, logdst, rows, slab,
               sent_src, sent_dst, sem):
        cid = lax.axis_index("c")
        sid = lax.axis_index("s")
        wid = sid * NC + cid
        base = wid * ROWS

        lanev = lax.iota(jnp.int32, 16)
        perms = [lanev ^ d for d in (1, 2, 4, 8)]

        sent_src[pl.ds(0, 16)] = jnp.broadcast_to(jnp.int32(SENT), (16,))
        sent_dst[pl.ds(0, 16)] = jnp.broadcast_to(jnp.int32(TRASH), (16,))

        zv = jnp.zeros((16,), jnp.float32)

        def zbody(r, carry):
            for j in range(D // 16):
                slab[r, pl.ds(j * 16, 16)] = zv
            return carry

        lax.fori_loop(0, SLAB, zbody, 0)

        def superchunk(sci, carry):
            e0 = sci * SCE
            pltpu.sync_copy(src_hbm.at[pl.ds(e0, SCE)], srcbuf)
            pltpu.sync_copy(dst_hbm.at[pl.ds(e0, SCE)], dstbuf)

            def chunk(k, cnt):
                vd = dstbuf[pl.ds(k * 16, 16)]
                vs = srcbuf[pl.ds(k * 16, 16)]
                u = vd - base
                inb = (u >= 0) & (u < ROWS)
                w0 = jnp.where(inb, u * 16 + lanev, BIG)
                n16 = _bsum(jnp.where(inb, 1, 0), perms)[0]

                def ext(i, carry2):
                    w, c = carry2
                    kminv = _bmin(w, perms)
                    lanemv = kminv & 15
                    logsrc[pl.ds(c, 16)] = _take16(vs, lanemv)
                    logdst[pl.ds(c, 16)] = kminv >> 4
                    w = jnp.where(lanev == lanemv, BIG, w)
                    return (w, c + 1)

                _, cnt = lax.fori_loop(0, n16, ext, (w0, cnt))
                return cnt

            cnt = lax.fori_loop(0, SCE // 16, chunk, jnp.int32(0))

            sv16 = sent_src[pl.ds(0, 16)]
            dv16 = sent_dst[pl.ds(0, 16)]
            for t in range(GB // 16):
                logsrc[pl.ds(cnt + 16 * t, 16)] = sv16
                logdst[pl.ds(cnt + 16 * t, 16)] = dv16

            nb = (cnt + (GB - 1)) // GB

            def batch(b, carry3):
                pltpu.async_copy(x_hbm.at[logsrc.at[pl.ds(b * GB, GB)]],
                                 rows, sem).wait()

                def acc(r, carry4):
                    dv = logdst[pl.ds(b * GB + r, 16)]
                    s = dv[0]
                    for j in range(D // 16):
                        slab[s, pl.ds(j * 16, 16)] = (
                            slab[s, pl.ds(j * 16, 16)]
                            + rows[r, pl.ds(j * 16, 16)])
                    return carry4

                lax.fori_loop(0, GB, acc, 0)
                return carry3

            lax.fori_loop(0, nb, batch, 0)
            return carry

        lax.fori_loop(0, NSC, superchunk, 0)

        pltpu.sync_copy(slab.at[pl.ds(0, ROWS)],
                        agg_hbm.at[pl.ds(base, ROWS)])

    return sc_agg


_sc_agg = _sc_agg_build()


def _dense_body(agg_ref, x_ref, wr1_ref, wo1_ref, wr2_ref, wo2_ref,
                b1_ref, b2_ref, out_ref, score_ref):
    a = agg_ref[...]
    xb = x_ref[...]
    dn = (((1,), (1,)), ((), ()))
    s1 = (lax.dot_general(a, wr1_ref[...], dn,
                          preferred_element_type=jnp.float32)
          + lax.dot_general(xb, wo1_ref[...], dn,
                            preferred_element_type=jnp.float32)
          + b1_ref[0, 0])
    x1 = jnp.tanh(s1)
    x2 = (lax.dot_general(a, wr2_ref[...], dn,
                          preferred_element_type=jnp.float32)
          + lax.dot_general(xb, wo2_ref[...], dn,
                            preferred_element_type=jnp.float32)
          + b2_ref[...])
    g = x1 * x2
    out_ref[...] = jnp.where(g > 0, g, jnp.exp(jnp.minimum(g, 0.0)) - 1.0)
    score_ref[...] = x1


def _dense(x, agg, W_rel1, W_root1, W_rel2, W_root2, b1, b2):
    BN = 1000
    grid = (N_NODES // BN,)
    return pl.pallas_call(
        _dense_body,
        grid=grid,
        in_specs=[
            pl.BlockSpec((BN, D), lambda i: (i, 0)),
            pl.BlockSpec((BN, D), lambda i: (i, 0)),
            pl.BlockSpec((1, D), lambda i: (0, 0)),
            pl.BlockSpec((1, D), lambda i: (0, 0)),
            pl.BlockSpec((D, D), lambda i: (0, 0)),
            pl.BlockSpec((D, D), lambda i: (0, 0)),
            pl.BlockSpec((1, 1), lambda i: (0, 0)),
            pl.BlockSpec((1, D), lambda i: (0, 0)),
        ],
        out_specs=[
            pl.BlockSpec((BN, D), lambda i: (i, 0)),
            pl.BlockSpec((BN, 1), lambda i: (i, 0)),
        ],
        out_shape=[
            jax.ShapeDtypeStruct((N_NODES, D), jnp.float32),
            jax.ShapeDtypeStruct((N_NODES, 1), jnp.float32),
        ],
    )(agg, x, W_rel1, W_root1, W_rel2, W_root2, b1, b2)


def kernel(x, edge_index, W_rel1, b_rel1, W_root1, W_rel2, b_rel2, W_root2):
    src = edge_index[0].astype(jnp.int32)
    dst = edge_index[1].astype(jnp.int32)
    x_pad = jnp.concatenate([x, jnp.zeros((1, D), jnp.float32)], axis=0)
    agg = _sc_agg(x_pad, src, dst)[:N_NODES]
    b1 = b_rel1.reshape(1, 1).astype(jnp.float32)
    b2 = b_rel2.reshape(1, D).astype(jnp.float32)
    out, score = _dense(x, agg, W_rel1, W_root1, W_rel2, W_root2, b1, b2)
    return out, score.reshape(-1)
